# baseline (speedup 1.0000x reference)
import functools

import jax
import jax.numpy as jnp
import numpy as np
from jax import lax
from jax.experimental import pallas as pl
from jax.experimental.pallas import tpu as pltpu
from jax.experimental.pallas import tpu_sc as plsc


def _gather_sparsecore(idx_flat, flat_table):
    n = idx_flat.shape[0]
    info = plsc.get_sparse_core_info()
    nc, ns = info.num_cores, info.num_subcores
    nw = nc * ns
    n_per_w = n // nw

    mesh = plsc.VectorSubcoreMesh(core_axis_name="c", subcore_axis_name="s")

    @functools.partial(
        pl.kernel,
        mesh=mesh,
        out_type=jax.ShapeDtypeStruct((n,), jnp.float32),
        scratch_types=[
            pltpu.VMEM((n_per_w,), jnp.int32),
            pltpu.VMEM((n_per_w,), jnp.float32),
            pltpu.SemaphoreType.DMA,
        ],
    )
    def gather_kernel(idx_hbm, table_hbm, out_hbm, idx_v, w_v, sem):
        wid = lax.axis_index("s") * nc + lax.axis_index("c")
        base = wid * n_per_w
        pltpu.sync_copy(idx_hbm.at[pl.ds(base, n_per_w)], idx_v)
        pltpu.async_copy(table_hbm.at[idx_v], w_v, sem).wait()
        pltpu.sync_copy(w_v, out_hbm.at[pl.ds(base, n_per_w)])

    return gather_kernel(idx_flat, flat_table)


def _interaction_tc(emb2d, sel):
    batch, fd = emb2d.shape
    d = sel.shape[1]
    blk = 1024
    rpb = blk // 128

    def body(emb_ref, sel_ref, out_ref):
        x = emb_ref[...]
        xf = x.astype(jnp.float32)
        sum_sq = jnp.sum(xf * xf, axis=1)
        s = jnp.dot(x, sel_ref[...],
                    preferred_element_type=jnp.float32)
        dots = 0.5 * (jnp.sum(s * s, axis=1) - sum_sq)
        out_ref[...] = dots.reshape(rpb, 128)

    return pl.pallas_call(
        body,
        grid=(batch // blk,),
        in_specs=[
            pl.BlockSpec((blk, fd), lambda i: (i, 0)),
            pl.BlockSpec((fd, d), lambda i: (0, 0)),
        ],
        out_specs=pl.BlockSpec((rpb, 128), lambda i: (i, 0)),
        out_shape=jax.ShapeDtypeStruct((batch // 128, 128), jnp.float32),
    )(emb2d, sel)


def _combine_tc(dots128, w_wide, k_sel, bias11):
    rows, wide = w_wide.shape
    rpb = 8

    def body(d_ref, w_ref, k_ref, bias_ref, out_ref):
        lr = jnp.dot(w_ref[...], k_ref[...],
                     preferred_element_type=jnp.float32)
        out_ref[...] = d_ref[...] + lr + bias_ref[0, 0]

    return pl.pallas_call(
        body,
        grid=(rows // rpb,),
        in_specs=[
            pl.BlockSpec((rpb, 128), lambda i: (i, 0)),
            pl.BlockSpec((rpb, wide), lambda i: (i, 0)),
            pl.BlockSpec((wide, 128), lambda i: (0, 0)),
            pl.BlockSpec((1, 1), lambda i: (0, 0)),
        ],
        out_specs=pl.BlockSpec((rpb, 128), lambda i: (i, 0)),
        out_shape=jax.ShapeDtypeStruct((rows, 128), jnp.float32),
    )(dots128, w_wide, k_sel, bias11)


def kernel(X, feature_emb, lr_table, bias):
    batch, nfields = X.shape
    vocab = lr_table.shape[1]
    d = feature_emb.shape[2]
    fd = nfields * d

    n = batch * nfields
    field_off = (np.arange(n, dtype=np.int32) % nfields) * vocab
    idx_flat = X.reshape(-1) + jnp.asarray(field_off)
    flat_table = lr_table.reshape(-1)

    w_flat = _gather_sparsecore(idx_flat, flat_table)

    sel = jnp.asarray(
        np.tile(np.eye(d, dtype=np.float32), (nfields, 1))
        .astype(np.float32)).astype(jnp.bfloat16)
    wide = 128 * nfields
    k_sel = jnp.asarray(
        (np.arange(wide)[:, None] // nfields
         == np.arange(128)[None, :]).astype(np.float32))

    emb_bf = feature_emb.astype(jnp.bfloat16).reshape(batch, fd)
    dots128 = _interaction_tc(emb_bf, sel)
    out128 = _combine_tc(dots128, w_flat.reshape(batch // 128, wide),
                         k_sel, bias.reshape(1, 1))
    return out128.reshape(batch, 1)


# speedup vs baseline: 1.0739x; 1.0739x over previous
import functools

import jax
import jax.numpy as jnp
import numpy as np
from jax import lax
from jax.experimental import pallas as pl
from jax.experimental.pallas import tpu as pltpu
from jax.experimental.pallas import tpu_sc as plsc


def _gather_sparsecore(idx_flat, flat_table):
    n = idx_flat.shape[0]
    info = plsc.get_sparse_core_info()
    nc, ns = info.num_cores, info.num_subcores
    nw = nc * ns
    n_per_w = n // nw

    mesh = plsc.VectorSubcoreMesh(core_axis_name="c", subcore_axis_name="s")

    @functools.partial(
        pl.kernel,
        mesh=mesh,
        out_type=jax.ShapeDtypeStruct((n,), jnp.float32),
        scratch_types=[
            pltpu.VMEM((n_per_w,), jnp.int32),
            pltpu.VMEM((n_per_w,), jnp.float32),
            pltpu.SemaphoreType.DMA,
        ],
    )
    def gather_kernel(idx_hbm, table_hbm, out_hbm, idx_v, w_v, sem):
        wid = lax.axis_index("s") * nc + lax.axis_index("c")
        base = wid * n_per_w
        pltpu.sync_copy(idx_hbm.at[pl.ds(base, n_per_w)], idx_v)
        pltpu.async_copy(table_hbm.at[idx_v], w_v, sem).wait()
        pltpu.sync_copy(w_v, out_hbm.at[pl.ds(base, n_per_w)])

    return gather_kernel(idx_flat, flat_table)


def _interaction_tc(emb2d, sel):
    batch, fd = emb2d.shape
    d = sel.shape[1]
    blk = 1024
    rpb = blk // 128

    def body(emb_ref, sel_ref, out_ref):
        x = emb_ref[...]
        xf = x.astype(jnp.float32)
        sum_sq = jnp.sum(xf * xf, axis=1)
        s = jnp.dot(x, sel_ref[...],
                    preferred_element_type=jnp.float32)
        dots = 0.5 * (jnp.sum(s * s, axis=1) - sum_sq)
        out_ref[...] = dots.reshape(rpb, 128)

    return pl.pallas_call(
        body,
        grid=(batch // blk,),
        in_specs=[
            pl.BlockSpec((blk, fd), lambda i: (i, 0)),
            pl.BlockSpec((fd, d), lambda i: (0, 0)),
        ],
        out_specs=pl.BlockSpec((rpb, 128), lambda i: (i, 0)),
        out_shape=jax.ShapeDtypeStruct((batch // 128, 128), jnp.float32),
    )(emb2d, sel)


def _combine_tc(dots128, w_wide, k_sel, bias11):
    rows, wide = w_wide.shape
    rpb = 8

    def body(d_ref, w_ref, k_ref, bias_ref, out_ref):
        lr = jnp.dot(w_ref[...], k_ref[...],
                     preferred_element_type=jnp.float32)
        out_ref[...] = d_ref[...] + lr + bias_ref[0, 0]

    del rpb
    return pl.pallas_call(
        body,
        grid=(1,),
        in_specs=[
            pl.BlockSpec((rows, 128), lambda i: (0, 0)),
            pl.BlockSpec((rows, wide), lambda i: (0, 0)),
            pl.BlockSpec((wide, 128), lambda i: (0, 0)),
            pl.BlockSpec((1, 1), lambda i: (0, 0)),
        ],
        out_specs=pl.BlockSpec((rows, 128), lambda i: (0, 0)),
        out_shape=jax.ShapeDtypeStruct((rows, 128), jnp.float32),
    )(dots128, w_wide, k_sel, bias11)


def kernel(X, feature_emb, lr_table, bias):
    batch, nfields = X.shape
    vocab = lr_table.shape[1]
    d = feature_emb.shape[2]
    fd = nfields * d

    n = batch * nfields
    field_off = (np.arange(n, dtype=np.int32) % nfields) * vocab
    idx_flat = X.reshape(-1) + jnp.asarray(field_off)

    w_flat = _gather_sparsecore(idx_flat, lr_table.reshape(-1))

    sel = jnp.asarray(
        np.tile(np.eye(d, dtype=np.float32), (nfields, 1))
        .astype(np.float32)).astype(jnp.bfloat16)
    wide = 128 * nfields
    k_sel = jnp.asarray(
        (np.arange(wide)[:, None] // nfields
         == np.arange(128)[None, :]).astype(np.float32))

    emb_bf = feature_emb.reshape(batch, fd).astype(jnp.bfloat16)
    dots128 = _interaction_tc(emb_bf, sel)
    out128 = _combine_tc(dots128, w_flat.reshape(batch // 128, wide),
                         k_sel, bias.reshape(1, 1))
    return out128.reshape(batch, 1)
